# baseline (speedup 1.0000x reference)
import functools

import jax
import jax.numpy as jnp
import numpy as np
from jax.experimental import pallas as pl
from jax.experimental.pallas import tpu as pltpu

_C = 96
_E = 5
_H = 224
_AD = (3, 0, 1, 6, 4, 2, 7, 8, 5)
_ROWS = 16


def _prep_kernel(cd_ref, ad_ref, hd_ref, vd_ref, std_ref, b_ref,
                 wext_ref, bext_ref):
    cd = cd_ref[...]
    ad = ad_ref[...]
    hd = hd_ref[...]
    vd = vd_ref[...]
    st = std_ref[...]
    s = jnp.sum(cd, axis=0)
    taps = []
    for t in range(9):
        kh, kw = divmod(t, 3)
        w = cd[t] + ad[t] - ad[_AD[t]] + st[t]
        if t == 4:
            w = w - s
        if kw == 0:
            w = w + hd[kh]
        elif kw == 2:
            w = w - hd[kh]
        if kh == 0:
            w = w + vd[kw]
        elif kh == 2:
            w = w - vd[kw]
        taps.append(w)
    wext_ref[...] = jnp.concatenate(taps, axis=0)
    bext_ref[...] = jnp.sum(b_ref[...], axis=0, keepdims=True)


def _fused_kernel(x_ref, wext_ref, bext_ref, wr_ref, brt_ref, tri_ref,
                  p2_ref, w1s_ref, w2s_ref, b2_ref, out_ref,
                  xp_ref, *, rows):
    i = pl.program_id(0)
    t_cnt = rows * _H
    n_blk = _H // rows

    @pl.when(i == 0)
    def _init_borders():
        xp_ref[0:1] = jnp.zeros((1, _H + 2, _C), dtype=jnp.float32)
        xp_ref[_H + 1:_H + 2] = jnp.zeros((1, _H + 2, _C), dtype=jnp.float32)
        xp_ref[:, 0:1, :] = jnp.zeros((_H + 2, 1, _C), dtype=jnp.float32)
        xp_ref[:, _H + 1:_H + 2, :] = jnp.zeros((_H + 2, 1, _C),
                                                dtype=jnp.float32)

    @pl.when(i < n_blk)
    def _stage_rows():
        xp_ref[pl.ds(1 + i * rows, rows), 1:_H + 1, :] = x_ref[0]

    @pl.when(i > 0)
    def _compute():
        j = i - 1
        xs = xp_ref[pl.ds(j * rows, rows + 2)]
        pieces = []
        for kh in range(3):
            band = xs[kh:kh + rows]
            for kw in range(3):
                pieces.append(band[:, kw:kw + _H, :])
        cat = jnp.concatenate(pieces, axis=-1).reshape(t_cnt, 9 * _C)
        y = jnp.dot(cat, wext_ref[...], preferred_element_type=jnp.float32)
        y = y + bext_ref[...]
        logits = jnp.dot(y, wr_ref[...], preferred_element_type=jnp.float32)
        logits = logits + brt_ref[...]

        m = jnp.max(logits, axis=-1, keepdims=True)
        eq = jnp.where(logits == m, 1.0, 0.0)
        csum = jnp.dot(eq, tri_ref[...], preferred_element_type=jnp.float32)
        sel = eq * jnp.where(csum == 1.0, 1.0, 0.0)
        bm = jnp.dot(sel, p2_ref[...], preferred_element_type=jnp.float32)
        bias2 = jnp.dot(sel, b2_ref[...], preferred_element_type=jnp.float32)

        yb = y.astype(jnp.bfloat16)
        h = jnp.dot(yb, w1s_ref[...], preferred_element_type=jnp.float32)
        h = jnp.maximum(h + bm, 0.0).astype(jnp.bfloat16)
        o = jnp.dot(h, w2s_ref[...], preferred_element_type=jnp.float32)
        o = o + bias2
        o = jnp.where(o >= 0.0, o, 0.01 * o)
        out_ref[...] = o.reshape(1, rows, _H, _C)


def kernel(x, w_cd, b_cd, w_hd, b_hd, w_vd, b_vd, w_ad, b_ad, w_std, b_std,
           w_router, b_router, w1, b1, w2, b2):
    C = _C
    cd9 = w_cd.reshape(C, C, 9).transpose(2, 1, 0)
    ad9 = w_ad.reshape(C, C, 9).transpose(2, 1, 0)
    std9 = w_std.reshape(C, C, 9).transpose(2, 1, 0)
    hd3 = w_hd.transpose(2, 1, 0)
    vd3 = w_vd.transpose(2, 1, 0)
    b5 = jnp.stack([b_cd, b_hd, b_vd, b_ad, b_std], axis=0)
    brt = b_router.reshape(1, _E)
    wext, bext = pl.pallas_call(
        _prep_kernel,
        out_shape=(jax.ShapeDtypeStruct((9 * C, C), jnp.float32),
                   jax.ShapeDtypeStruct((1, C), jnp.float32)),
    )(cd9, ad9, hd3, vd3, std9, b5)

    w1s = w1.transpose(1, 0, 2).reshape(C, _E * C).astype(jnp.bfloat16)
    w2s = w2.reshape(_E * C, C).astype(jnp.bfloat16)
    tri = jnp.asarray(np.triu(np.ones((_E, _E), np.float32)))
    pmask = jnp.asarray(np.kron(np.eye(_E, dtype=np.float32),
                                np.ones((1, C), np.float32)))
    b1rep = jnp.tile(b1.reshape(_E, C), (1, _E))
    p2 = pmask * b1rep - (1.0 - pmask) * 1e30

    rows = _ROWS
    n_blk = _H // rows
    out = pl.pallas_call(
        functools.partial(_fused_kernel, rows=rows),
        grid=(n_blk + 1,),
        in_specs=[
            pl.BlockSpec((1, rows, _H, C),
                         lambda i: (0, jnp.minimum(i, _H // _ROWS - 1), 0, 0)),
            pl.BlockSpec((9 * C, C), lambda i: (0, 0)),
            pl.BlockSpec((1, C), lambda i: (0, 0)),
            pl.BlockSpec((C, _E), lambda i: (0, 0)),
            pl.BlockSpec((1, _E), lambda i: (0, 0)),
            pl.BlockSpec((_E, _E), lambda i: (0, 0)),
            pl.BlockSpec((_E, _E * C), lambda i: (0, 0)),
            pl.BlockSpec((C, _E * C), lambda i: (0, 0)),
            pl.BlockSpec((_E * C, C), lambda i: (0, 0)),
            pl.BlockSpec((_E, C), lambda i: (0, 0)),
        ],
        out_specs=pl.BlockSpec((1, rows, _H, C),
                               lambda i: (0, jnp.maximum(i - 1, 0), 0, 0)),
        out_shape=jax.ShapeDtypeStruct((1, _H, _H, C), jnp.float32),
        scratch_shapes=[pltpu.VMEM((_H + 2, _H + 2, C), jnp.float32)],
        compiler_params=pltpu.CompilerParams(
            vmem_limit_bytes=100 * 1024 * 1024),
    )(x, wext, bext, w_router, brt, tri, p2, w1s, w2s, b2)
    return out


# speedup vs baseline: 1.0045x; 1.0045x over previous
import functools

import jax
import jax.numpy as jnp
import numpy as np
from jax.experimental import pallas as pl
from jax.experimental.pallas import tpu as pltpu

_C = 96
_E = 5
_H = 224
_AD = (3, 0, 1, 6, 4, 2, 7, 8, 5)
_ROWS = 28


def _prep_kernel(cd_ref, ad_ref, hd_ref, vd_ref, std_ref, b_ref,
                 wext_ref, bext_ref):
    cd = cd_ref[...]
    ad = ad_ref[...]
    hd = hd_ref[...]
    vd = vd_ref[...]
    st = std_ref[...]
    s = jnp.sum(cd, axis=0)
    taps = []
    for t in range(9):
        kh, kw = divmod(t, 3)
        w = cd[t] + ad[t] - ad[_AD[t]] + st[t]
        if t == 4:
            w = w - s
        if kw == 0:
            w = w + hd[kh]
        elif kw == 2:
            w = w - hd[kh]
        if kh == 0:
            w = w + vd[kw]
        elif kh == 2:
            w = w - vd[kw]
        taps.append(w)
    wext_ref[...] = jnp.concatenate(taps, axis=0)
    bext_ref[...] = jnp.sum(b_ref[...], axis=0, keepdims=True)


def _fused_kernel(x_ref, wext_ref, bext_ref, wr_ref, brt_ref, tri_ref,
                  p2_ref, w1s_ref, w2s_ref, b2_ref, out_ref,
                  xp_ref, *, rows):
    i = pl.program_id(0)
    t_cnt = rows * _H
    n_blk = _H // rows

    @pl.when(i == 0)
    def _init_borders():
        xp_ref[0:1] = jnp.zeros((1, _H + 2, _C), dtype=jnp.float32)
        xp_ref[_H + 1:_H + 2] = jnp.zeros((1, _H + 2, _C), dtype=jnp.float32)
        xp_ref[:, 0:1, :] = jnp.zeros((_H + 2, 1, _C), dtype=jnp.float32)
        xp_ref[:, _H + 1:_H + 2, :] = jnp.zeros((_H + 2, 1, _C),
                                                dtype=jnp.float32)

    @pl.when(i < n_blk)
    def _stage_rows():
        xp_ref[pl.ds(1 + i * rows, rows), 1:_H + 1, :] = x_ref[0]

    @pl.when(i > 0)
    def _compute():
        j = i - 1
        xs = xp_ref[pl.ds(j * rows, rows + 2)]
        pieces = []
        for kh in range(3):
            band = xs[kh:kh + rows]
            for kw in range(3):
                pieces.append(band[:, kw:kw + _H, :])
        cat = jnp.concatenate(pieces, axis=-1).reshape(t_cnt, 9 * _C)
        y = jnp.dot(cat, wext_ref[...], preferred_element_type=jnp.float32)
        y = y + bext_ref[...]
        logits = jnp.dot(y, wr_ref[...], preferred_element_type=jnp.float32)
        logits = logits + brt_ref[...]

        m = jnp.max(logits, axis=-1, keepdims=True)
        eq = jnp.where(logits == m, 1.0, 0.0)
        csum = jnp.dot(eq, tri_ref[...], preferred_element_type=jnp.float32)
        sel = eq * jnp.where(csum == 1.0, 1.0, 0.0)
        bm = jnp.dot(sel, p2_ref[...], preferred_element_type=jnp.float32)
        bias2 = jnp.dot(sel, b2_ref[...], preferred_element_type=jnp.float32)

        yb = y.astype(jnp.bfloat16)
        h = jnp.dot(yb, w1s_ref[...], preferred_element_type=jnp.float32)
        h = jnp.maximum(h + bm, 0.0).astype(jnp.bfloat16)
        o = jnp.dot(h, w2s_ref[...], preferred_element_type=jnp.float32)
        o = o + bias2
        o = jnp.where(o >= 0.0, o, 0.01 * o)
        out_ref[...] = o.reshape(1, rows, _H, _C)


def kernel(x, w_cd, b_cd, w_hd, b_hd, w_vd, b_vd, w_ad, b_ad, w_std, b_std,
           w_router, b_router, w1, b1, w2, b2):
    C = _C
    cd9 = w_cd.reshape(C, C, 9).transpose(2, 1, 0)
    ad9 = w_ad.reshape(C, C, 9).transpose(2, 1, 0)
    std9 = w_std.reshape(C, C, 9).transpose(2, 1, 0)
    hd3 = w_hd.transpose(2, 1, 0)
    vd3 = w_vd.transpose(2, 1, 0)
    b5 = jnp.stack([b_cd, b_hd, b_vd, b_ad, b_std], axis=0)
    brt = b_router.reshape(1, _E)
    wext, bext = pl.pallas_call(
        _prep_kernel,
        out_shape=(jax.ShapeDtypeStruct((9 * C, C), jnp.float32),
                   jax.ShapeDtypeStruct((1, C), jnp.float32)),
    )(cd9, ad9, hd3, vd3, std9, b5)

    w1s = w1.transpose(1, 0, 2).reshape(C, _E * C).astype(jnp.bfloat16)
    w2s = w2.reshape(_E * C, C).astype(jnp.bfloat16)
    tri = jnp.asarray(np.triu(np.ones((_E, _E), np.float32)))
    pmask = jnp.asarray(np.kron(np.eye(_E, dtype=np.float32),
                                np.ones((1, C), np.float32)))
    b1rep = jnp.tile(b1.reshape(_E, C), (1, _E))
    p2 = pmask * b1rep - (1.0 - pmask) * 1e30

    rows = _ROWS
    n_blk = _H // rows
    out = pl.pallas_call(
        functools.partial(_fused_kernel, rows=rows),
        grid=(n_blk + 1,),
        in_specs=[
            pl.BlockSpec((1, rows, _H, C),
                         lambda i: (0, jnp.minimum(i, _H // _ROWS - 1), 0, 0)),
            pl.BlockSpec((9 * C, C), lambda i: (0, 0)),
            pl.BlockSpec((1, C), lambda i: (0, 0)),
            pl.BlockSpec((C, _E), lambda i: (0, 0)),
            pl.BlockSpec((1, _E), lambda i: (0, 0)),
            pl.BlockSpec((_E, _E), lambda i: (0, 0)),
            pl.BlockSpec((_E, _E * C), lambda i: (0, 0)),
            pl.BlockSpec((C, _E * C), lambda i: (0, 0)),
            pl.BlockSpec((_E * C, C), lambda i: (0, 0)),
            pl.BlockSpec((_E, C), lambda i: (0, 0)),
        ],
        out_specs=pl.BlockSpec((1, rows, _H, C),
                               lambda i: (0, jnp.maximum(i - 1, 0), 0, 0)),
        out_shape=jax.ShapeDtypeStruct((1, _H, _H, C), jnp.float32),
        scratch_shapes=[pltpu.VMEM((_H + 2, _H + 2, C), jnp.float32)],
        compiler_params=pltpu.CompilerParams(
            vmem_limit_bytes=100 * 1024 * 1024),
    )(x, wext, bext, w_router, brt, tri, p2, w1s, w2s, b2)
    return out


# speedup vs baseline: 1.0142x; 1.0097x over previous
import functools

import jax
import jax.numpy as jnp
import numpy as np
from jax.experimental import pallas as pl
from jax.experimental.pallas import tpu as pltpu

_C = 96
_E = 5
_H = 224
_AD = (3, 0, 1, 6, 4, 2, 7, 8, 5)
_ROWS = 28


def _fused_kernel(x_ref, cd_ref, ad_ref, hd_ref, vd_ref, std_ref, b_ref,
                  wr_ref, brt_ref, tri_ref,
                  p2_ref, w1s_ref, w2s_ref, b2_ref, out_ref,
                  xp_ref, wext_ref, bext_ref, *, rows):
    i = pl.program_id(0)
    t_cnt = rows * _H
    n_blk = _H // rows

    @pl.when(i == 0)
    def _init_borders():
        xp_ref[0:1] = jnp.zeros((1, _H + 2, _C), dtype=jnp.float32)
        xp_ref[_H + 1:_H + 2] = jnp.zeros((1, _H + 2, _C), dtype=jnp.float32)
        xp_ref[:, 0:1, :] = jnp.zeros((_H + 2, 1, _C), dtype=jnp.float32)
        xp_ref[:, _H + 1:_H + 2, :] = jnp.zeros((_H + 2, 1, _C),
                                                dtype=jnp.float32)
        cd = cd_ref[...]
        ad = ad_ref[...]
        hd = hd_ref[...]
        vd = vd_ref[...]
        st = std_ref[...]
        s = jnp.sum(cd, axis=0)
        taps = []
        for t in range(9):
            kh, kw = divmod(t, 3)
            w = cd[t] + ad[t] - ad[_AD[t]] + st[t]
            if t == 4:
                w = w - s
            if kw == 0:
                w = w + hd[kh]
            elif kw == 2:
                w = w - hd[kh]
            if kh == 0:
                w = w + vd[kw]
            elif kh == 2:
                w = w - vd[kw]
            taps.append(w)
        wext_ref[...] = jnp.concatenate(taps, axis=0)
        bext_ref[...] = jnp.sum(b_ref[...], axis=0, keepdims=True)

    @pl.when(i < n_blk)
    def _stage_rows():
        xp_ref[pl.ds(1 + i * rows, rows), 1:_H + 1, :] = x_ref[0]

    @pl.when(i > 0)
    def _compute():
        j = i - 1
        xs = xp_ref[pl.ds(j * rows, rows + 2)]
        pieces = []
        for kh in range(3):
            band = xs[kh:kh + rows]
            for kw in range(3):
                pieces.append(band[:, kw:kw + _H, :])
        cat = jnp.concatenate(pieces, axis=-1).reshape(t_cnt, 9 * _C)
        y = jnp.dot(cat, wext_ref[...], preferred_element_type=jnp.float32)
        y = y + bext_ref[...]
        logits = jnp.dot(y, wr_ref[...], preferred_element_type=jnp.float32)
        logits = logits + brt_ref[...]

        m = jnp.max(logits, axis=-1, keepdims=True)
        eq = jnp.where(logits == m, 1.0, 0.0)
        csum = jnp.dot(eq, tri_ref[...], preferred_element_type=jnp.float32)
        sel = eq * jnp.where(csum == 1.0, 1.0, 0.0)
        bm = jnp.dot(sel, p2_ref[...], preferred_element_type=jnp.float32)
        bias2 = jnp.dot(sel, b2_ref[...], preferred_element_type=jnp.float32)

        yb = y.astype(jnp.bfloat16)
        h = jnp.dot(yb, w1s_ref[...], preferred_element_type=jnp.float32)
        h = jnp.maximum(h + bm, 0.0).astype(jnp.bfloat16)
        o = jnp.dot(h, w2s_ref[...], preferred_element_type=jnp.float32)
        o = o + bias2
        o = jnp.where(o >= 0.0, o, 0.01 * o)
        out_ref[...] = o.reshape(1, rows, _H, _C)


def kernel(x, w_cd, b_cd, w_hd, b_hd, w_vd, b_vd, w_ad, b_ad, w_std, b_std,
           w_router, b_router, w1, b1, w2, b2):
    C = _C
    cd9 = w_cd.reshape(C, C, 9).transpose(2, 1, 0)
    ad9 = w_ad.reshape(C, C, 9).transpose(2, 1, 0)
    std9 = w_std.reshape(C, C, 9).transpose(2, 1, 0)
    hd3 = w_hd.transpose(2, 1, 0)
    vd3 = w_vd.transpose(2, 1, 0)
    b5 = jnp.stack([b_cd, b_hd, b_vd, b_ad, b_std], axis=0)
    brt = b_router.reshape(1, _E)
    w1s = w1.transpose(1, 0, 2).reshape(C, _E * C).astype(jnp.bfloat16)
    w2s = w2.reshape(_E * C, C).astype(jnp.bfloat16)
    tri = jnp.asarray(np.triu(np.ones((_E, _E), np.float32)))
    pmask = jnp.asarray(np.kron(np.eye(_E, dtype=np.float32),
                                np.ones((1, C), np.float32)))
    b1rep = jnp.tile(b1.reshape(_E, C), (1, _E))
    p2 = pmask * b1rep - (1.0 - pmask) * 1e30

    rows = _ROWS
    n_blk = _H // rows
    out = pl.pallas_call(
        functools.partial(_fused_kernel, rows=rows),
        grid=(n_blk + 1,),
        in_specs=[
            pl.BlockSpec((1, rows, _H, C),
                         lambda i: (0, jnp.minimum(i, _H // _ROWS - 1), 0, 0)),
            pl.BlockSpec((9, C, C), lambda i: (0, 0, 0)),
            pl.BlockSpec((9, C, C), lambda i: (0, 0, 0)),
            pl.BlockSpec((3, C, C), lambda i: (0, 0, 0)),
            pl.BlockSpec((3, C, C), lambda i: (0, 0, 0)),
            pl.BlockSpec((9, C, C), lambda i: (0, 0, 0)),
            pl.BlockSpec((_E, C), lambda i: (0, 0)),
            pl.BlockSpec((C, _E), lambda i: (0, 0)),
            pl.BlockSpec((1, _E), lambda i: (0, 0)),
            pl.BlockSpec((_E, _E), lambda i: (0, 0)),
            pl.BlockSpec((_E, _E * C), lambda i: (0, 0)),
            pl.BlockSpec((C, _E * C), lambda i: (0, 0)),
            pl.BlockSpec((_E * C, C), lambda i: (0, 0)),
            pl.BlockSpec((_E, C), lambda i: (0, 0)),
        ],
        out_specs=pl.BlockSpec((1, rows, _H, C),
                               lambda i: (0, jnp.maximum(i - 1, 0), 0, 0)),
        out_shape=jax.ShapeDtypeStruct((1, _H, _H, C), jnp.float32),
        scratch_shapes=[pltpu.VMEM((_H + 2, _H + 2, _C), jnp.float32),
                        pltpu.VMEM((9 * _C, _C), jnp.float32),
                        pltpu.VMEM((1, _C), jnp.float32)],
        compiler_params=pltpu.CompilerParams(
            vmem_limit_bytes=100 * 1024 * 1024),
    )(x, cd9, ad9, hd3, vd3, std9, b5, w_router, brt, tri, p2, w1s, w2s, b2)
    return out


# speedup vs baseline: 1.1912x; 1.1744x over previous
import functools

import jax
import jax.numpy as jnp
import numpy as np
from jax.experimental import pallas as pl
from jax.experimental.pallas import tpu as pltpu

_C = 96
_E = 5
_H = 224
_AD = (3, 0, 1, 6, 4, 2, 7, 8, 5)
_ROWS = 28
_W = 232


def _fused_kernel(x_ref, cd_ref, ad_ref, hd_ref, vd_ref, std_ref, b_ref,
                  wr_ref, brt_ref, tri_ref,
                  p2_ref, w1s_ref, w2s_ref, b2_ref, out_ref,
                  xp_ref, w3_ref, bext_ref, *, rows):
    i = pl.program_id(0)
    t_cnt = rows * _H
    n_blk = _H // rows

    @pl.when(i == 0)
    def _init_borders():
        xp_ref[0:1] = jnp.zeros((1, _W, _C), dtype=jnp.float32)
        xp_ref[_H + 1:_H + 2] = jnp.zeros((1, _W, _C), dtype=jnp.float32)
        xp_ref[:, 0:1, :] = jnp.zeros((_H + 2, 1, _C), dtype=jnp.float32)
        xp_ref[:, _H + 1:_W, :] = jnp.zeros((_H + 2, _W - _H - 1, _C),
                                            dtype=jnp.float32)
        cd = cd_ref[...]
        ad = ad_ref[...]
        hd = hd_ref[...]
        vd = vd_ref[...]
        st = std_ref[...]
        s = jnp.sum(cd, axis=0)
        taps = []
        for t in range(9):
            kh, kw = divmod(t, 3)
            w = cd[t] + ad[t] - ad[_AD[t]] + st[t]
            if t == 4:
                w = w - s
            if kw == 0:
                w = w + hd[kh]
            elif kw == 2:
                w = w - hd[kh]
            if kh == 0:
                w = w + vd[kw]
            elif kh == 2:
                w = w - vd[kw]
            taps.append(w)
        zc = jnp.zeros((_C, 32), dtype=jnp.float32)
        zr = jnp.zeros((32, 384), dtype=jnp.float32)
        rows_w3 = []
        for kh in range(3):
            blk = jnp.concatenate(
                [jnp.concatenate([taps[kh * 3 + kw], zc], axis=1)
                 for kw in range(3)], axis=1)
            rows_w3.append(blk)
            rows_w3.append(zr)
        w3_ref[...] = jnp.concatenate(rows_w3, axis=0)
        bext_ref[...] = jnp.sum(b_ref[...], axis=0, keepdims=True)

    @pl.when(i < n_blk)
    def _stage_rows():
        xp_ref[pl.ds(1 + i * rows, rows), 1:_H + 1, :] = x_ref[0]

    @pl.when(i > 0)
    def _compute():
        j = i - 1
        tp = rows * _W
        xs = xp_ref[pl.ds(j * rows, rows + 2)]
        zl = jnp.zeros((tp, 32), dtype=jnp.float32)
        bandh = jnp.concatenate(
            [jnp.concatenate([xs[kh:kh + rows].reshape(tp, _C), zl], axis=1)
             for kh in range(3)], axis=1)
        p = jnp.dot(bandh, w3_ref[...], preferred_element_type=jnp.float32)
        ym = (p[0:tp - 2, 0:_C] + p[1:tp - 1, 128:128 + _C]
              + p[2:tp, 256:256 + _C])
        ym = jnp.concatenate([ym, jnp.zeros((2, _C), dtype=jnp.float32)],
                             axis=0)
        y = ym.reshape(rows, _W, _C)[:, :_H, :].reshape(t_cnt, _C)
        y = y + bext_ref[...]
        logits = jnp.dot(y, wr_ref[...], preferred_element_type=jnp.float32)
        logits = logits + brt_ref[...]

        m = jnp.max(logits, axis=-1, keepdims=True)
        eq = jnp.where(logits == m, 1.0, 0.0)
        csum = jnp.dot(eq, tri_ref[...], preferred_element_type=jnp.float32)
        sel = eq * jnp.where(csum == 1.0, 1.0, 0.0)
        bm = jnp.dot(sel, p2_ref[...], preferred_element_type=jnp.float32)
        bias2 = jnp.dot(sel, b2_ref[...], preferred_element_type=jnp.float32)

        yb = y.astype(jnp.bfloat16)
        h = jnp.dot(yb, w1s_ref[...], preferred_element_type=jnp.float32)
        h = jnp.maximum(h + bm, 0.0).astype(jnp.bfloat16)
        o = jnp.dot(h, w2s_ref[...], preferred_element_type=jnp.float32)
        o = o + bias2
        o = jnp.where(o >= 0.0, o, 0.01 * o)
        out_ref[...] = o.reshape(1, rows, _H, _C)


def kernel(x, w_cd, b_cd, w_hd, b_hd, w_vd, b_vd, w_ad, b_ad, w_std, b_std,
           w_router, b_router, w1, b1, w2, b2):
    C = _C
    cd9 = w_cd.reshape(C, C, 9).transpose(2, 1, 0)
    ad9 = w_ad.reshape(C, C, 9).transpose(2, 1, 0)
    std9 = w_std.reshape(C, C, 9).transpose(2, 1, 0)
    hd3 = w_hd.transpose(2, 1, 0)
    vd3 = w_vd.transpose(2, 1, 0)
    b5 = jnp.stack([b_cd, b_hd, b_vd, b_ad, b_std], axis=0)
    brt = b_router.reshape(1, _E)
    w1s = w1.transpose(1, 0, 2).reshape(C, _E * C).astype(jnp.bfloat16)
    w2s = w2.reshape(_E * C, C).astype(jnp.bfloat16)
    tri = jnp.asarray(np.triu(np.ones((_E, _E), np.float32)))
    pmask = jnp.asarray(np.kron(np.eye(_E, dtype=np.float32),
                                np.ones((1, C), np.float32)))
    b1rep = jnp.tile(b1.reshape(_E, C), (1, _E))
    p2 = pmask * b1rep - (1.0 - pmask) * 1e30

    rows = _ROWS
    n_blk = _H // rows
    out = pl.pallas_call(
        functools.partial(_fused_kernel, rows=rows),
        grid=(n_blk + 1,),
        in_specs=[
            pl.BlockSpec((1, rows, _H, C),
                         lambda i: (0, jnp.minimum(i, _H // _ROWS - 1), 0, 0)),
            pl.BlockSpec((9, C, C), lambda i: (0, 0, 0)),
            pl.BlockSpec((9, C, C), lambda i: (0, 0, 0)),
            pl.BlockSpec((3, C, C), lambda i: (0, 0, 0)),
            pl.BlockSpec((3, C, C), lambda i: (0, 0, 0)),
            pl.BlockSpec((9, C, C), lambda i: (0, 0, 0)),
            pl.BlockSpec((_E, C), lambda i: (0, 0)),
            pl.BlockSpec((C, _E), lambda i: (0, 0)),
            pl.BlockSpec((1, _E), lambda i: (0, 0)),
            pl.BlockSpec((_E, _E), lambda i: (0, 0)),
            pl.BlockSpec((_E, _E * C), lambda i: (0, 0)),
            pl.BlockSpec((C, _E * C), lambda i: (0, 0)),
            pl.BlockSpec((_E * C, C), lambda i: (0, 0)),
            pl.BlockSpec((_E, C), lambda i: (0, 0)),
        ],
        out_specs=pl.BlockSpec((1, rows, _H, C),
                               lambda i: (0, jnp.maximum(i - 1, 0), 0, 0)),
        out_shape=jax.ShapeDtypeStruct((1, _H, _H, C), jnp.float32),
        scratch_shapes=[pltpu.VMEM((_H + 2, _W, _C), jnp.float32),
                        pltpu.VMEM((384, 384), jnp.float32),
                        pltpu.VMEM((1, _C), jnp.float32)],
        compiler_params=pltpu.CompilerParams(
            vmem_limit_bytes=100 * 1024 * 1024),
    )(x, cd9, ad9, hd3, vd3, std9, b5, w_router, brt, tri, p2, w1s, w2s, b2)
    return out
